# Initial kernel scaffold; baseline (speedup 1.0000x reference)
import functools

import jax
import jax.numpy as jnp
from jax import lax
from jax.experimental import pallas as pl
from jax.experimental.pallas import tpu as pltpu
from jax.experimental.pallas import tpu_sc as plsc

N_NODES = 10000
D_AGG = 128
E_TOTAL = 320000

NC, NS = 2, 16
NW = NC * NS
EPW = E_TOTAL // NW
CHUNK = 80
NCHUNK = EPW // CHUNK
ROWS_PT = N_NODES // NS

_MESH = plsc.VectorSubcoreMesh(core_axis_name="c", subcore_axis_name="s")


def _deg_body(dst_h, ew_h, out_h, dstv, ewv, zer, acc):
    c = lax.axis_index("c")
    s = lax.axis_index("s")
    wid = s * NC + c

    def zinit(i, carry):
        zer[pl.ds(i * 16, 16)] = jnp.zeros((16,), jnp.float32)
        return carry

    lax.fori_loop(0, 125, zinit, 0)

    @pl.when(s < 5)
    def _():
        pltpu.sync_copy(zer, acc.at[pl.ds(s * 2000, 2000)])

    plsc.subcore_barrier()

    ebase = wid * EPW

    def chunk_body(k, carry):
        off = ebase + k * CHUNK
        pltpu.sync_copy(dst_h.at[pl.ds(off, CHUNK)], dstv)
        pltpu.sync_copy(ew_h.at[pl.ds(off, CHUNK)], ewv)
        pltpu.sync_copy(ewv, acc.at[dstv], add=True)
        return carry

    lax.fori_loop(0, NCHUNK, chunk_body, 0)
    plsc.subcore_barrier()

    @pl.when(s == 0)
    def _():
        pltpu.sync_copy(acc, out_h.at[pl.ds(c * N_NODES, N_NODES)])


_deg_call = functools.partial(
    pl.kernel,
    out_type=jax.ShapeDtypeStruct((2 * N_NODES,), jnp.float32),
    mesh=_MESH,
    scratch_types=[
        pltpu.VMEM((CHUNK,), jnp.int32),
        pltpu.VMEM((CHUNK,), jnp.float32),
        pltpu.VMEM((2000,), jnp.float32),
        pltpu.VMEM_SHARED((N_NODES,), jnp.float32),
    ],
)(_deg_body)


def _spmm_body(src_h, dst_h, ew_h, xs_h, out_h, srcv, dstv, ewv, rows, sem, acc):
    c = lax.axis_index("c")
    s = lax.axis_index("s")
    wid = s * NC + c

    def zrow(e, carry):
        for j in range(8):
            rows[e, pl.ds(j * 16, 16)] = jnp.zeros((16,), jnp.float32)
        return carry

    lax.fori_loop(0, CHUNK, zrow, 0)
    base_r = s * ROWS_PT
    for k in range(ROWS_PT // CHUNK):
        pltpu.sync_copy(rows, acc.at[pl.ds(base_r + k * CHUNK, CHUNK)])
    rem = ROWS_PT % CHUNK
    if rem:
        pltpu.sync_copy(
            rows.at[pl.ds(0, rem)],
            acc.at[pl.ds(base_r + (ROWS_PT // CHUNK) * CHUNK, rem)],
        )
    plsc.subcore_barrier()

    ebase = wid * EPW

    def chunk_body(k, carry):
        off = ebase + k * CHUNK
        pltpu.sync_copy(src_h.at[pl.ds(off, CHUNK)], srcv)
        pltpu.sync_copy(dst_h.at[pl.ds(off, CHUNK)], dstv)
        pltpu.sync_copy(ew_h.at[pl.ds(off, CHUNK)], ewv)
        pltpu.async_copy(xs_h.at[srcv], rows, sem).wait()

        def edge_body(e, inner):
            w = ewv[e]
            for j in range(8):
                rows[e, pl.ds(j * 16, 16)] = rows[e, pl.ds(j * 16, 16)] * w
            return inner

        lax.fori_loop(0, CHUNK, edge_body, 0)
        pltpu.sync_copy(rows, acc.at[dstv], add=True)
        return carry

    lax.fori_loop(0, NCHUNK, chunk_body, 0)
    plsc.subcore_barrier()

    out_base = c * N_NODES + s * ROWS_PT
    pltpu.sync_copy(
        acc.at[pl.ds(s * ROWS_PT, ROWS_PT)], out_h.at[pl.ds(out_base, ROWS_PT)]
    )


_spmm_call = functools.partial(
    pl.kernel,
    out_type=jax.ShapeDtypeStruct((2 * N_NODES, D_AGG), jnp.float32),
    mesh=_MESH,
    scratch_types=[
        pltpu.VMEM((CHUNK,), jnp.int32),
        pltpu.VMEM((CHUNK,), jnp.int32),
        pltpu.VMEM((CHUNK,), jnp.float32),
        pltpu.VMEM((CHUNK, D_AGG), jnp.float32),
        pltpu.SemaphoreType.DMA,
        pltpu.VMEM_SHARED((N_NODES, D_AGG), jnp.float32),
    ],
)(_spmm_body)


_BLK = 1000
_GRID = N_NODES // _BLK


def _dinv_block(p0, p1):
    deg = p0 + p1 + 1.0
    return jnp.where(deg > 0, lax.rsqrt(deg), 0.0)


def _prep_body(p0_ref, p1_ref, x_ref, xs_ref):
    dinv = _dinv_block(p0_ref[...], p1_ref[...])
    xs_ref[...] = x_ref[...] * dinv


def _layer1_body(z0_ref, z1_ref, xs_ref, p0_ref, p1_ref, w1_ref, b1_ref,
                 w2_ref, out_ref):
    dinv = _dinv_block(p0_ref[...], p1_ref[...])
    y1 = (z0_ref[...] + z1_ref[...] + xs_ref[...]) * dinv
    h1 = jnp.maximum(
        jnp.dot(y1, w1_ref[...], preferred_element_type=jnp.float32)
        + b1_ref[...],
        0.0,
    )
    out_ref[...] = (
        jnp.dot(h1, w2_ref[...], preferred_element_type=jnp.float32) * dinv
    )


def _layer2_body(z0_ref, z1_ref, xs_ref, p0_ref, p1_ref, b2_ref, wfc_ref,
                 bfc_ref, out_ref):
    dinv = _dinv_block(p0_ref[...], p1_ref[...])
    y2 = (z0_ref[...] + z1_ref[...] + xs_ref[...]) * dinv
    h2 = jnp.maximum(y2 + b2_ref[...], 0.0)
    out_ref[...] = (
        jnp.dot(h2, wfc_ref[...], preferred_element_type=jnp.float32)
        + bfc_ref[...]
    )


def _row_spec(width):
    return pl.BlockSpec((_BLK, width), lambda i: (i, 0))


def _full_spec(shape):
    return pl.BlockSpec(shape, lambda i: (0, 0))


_prep_call = pl.pallas_call(
    _prep_body,
    grid=(_GRID,),
    in_specs=[_row_spec(1), _row_spec(1), _row_spec(D_AGG)],
    out_specs=_row_spec(D_AGG),
    out_shape=jax.ShapeDtypeStruct((N_NODES, D_AGG), jnp.float32),
)

_layer1_call = pl.pallas_call(
    _layer1_body,
    grid=(_GRID,),
    in_specs=[
        _row_spec(D_AGG), _row_spec(D_AGG), _row_spec(D_AGG),
        _row_spec(1), _row_spec(1),
        _full_spec((128, 256)), _full_spec((1, 256)), _full_spec((256, 128)),
    ],
    out_specs=_row_spec(D_AGG),
    out_shape=jax.ShapeDtypeStruct((N_NODES, D_AGG), jnp.float32),
)

_layer2_call = pl.pallas_call(
    _layer2_body,
    grid=(_GRID,),
    in_specs=[
        _row_spec(D_AGG), _row_spec(D_AGG), _row_spec(D_AGG),
        _row_spec(1), _row_spec(1),
        _full_spec((1, 128)), _full_spec((128, 1)), _full_spec((1, 1)),
    ],
    out_specs=_row_spec(1),
    out_shape=jax.ShapeDtypeStruct((N_NODES, 1), jnp.float32),
)


def kernel(x, edge_index, edge_attr, W1, b1, W2, b2, Wfc, bfc):
    src = edge_index[0]
    dst = edge_index[1]
    ew = edge_attr

    degp = _deg_call(dst, ew)
    p0 = degp[:N_NODES].reshape(N_NODES, 1)
    p1 = degp[N_NODES:].reshape(N_NODES, 1)

    xs1 = _prep_call(p0, p1, x)
    z = _spmm_call(src, dst, ew, xs1)
    xs2 = _layer1_call(
        z[:N_NODES], z[N_NODES:], xs1, p0, p1, W1, b1.reshape(1, -1), W2
    )
    z2 = _spmm_call(src, dst, ew, xs2)
    out = _layer2_call(
        z2[:N_NODES], z2[N_NODES:], xs2, p0, p1,
        b2.reshape(1, -1), Wfc, bfc.reshape(1, 1),
    )
    return out


# speedup vs baseline: 10.7059x; 10.7059x over previous
import functools

import jax
import jax.numpy as jnp
from jax import lax
from jax.experimental import pallas as pl
from jax.experimental.pallas import tpu as pltpu
from jax.experimental.pallas import tpu_sc as plsc

N_NODES = 10000
D_AGG = 128
E_TOTAL = 320000

NC, NS = 2, 16
NW = NC * NS
EPW = E_TOTAL // NW
CHUNK = 80
NCHUNK = EPW // CHUNK
ROWS_PT = N_NODES // NS

_MESH = plsc.VectorSubcoreMesh(core_axis_name="c", subcore_axis_name="s")


def _deg_body(dst_h, ew_h, out_h, dstv, ewv, zer, acc):
    c = lax.axis_index("c")
    s = lax.axis_index("s")
    wid = s * NC + c

    def zinit(i, carry):
        zer[pl.ds(pl.multiple_of(i * 16, 16), 16)] = jnp.zeros((16,), jnp.float32)
        return carry

    lax.fori_loop(0, 128, zinit, 0)

    @pl.when(s < 4)
    def _():
        pltpu.sync_copy(zer, acc.at[pl.ds(s * 2048, 2048)])

    @pl.when(s == 4)
    def _():
        pltpu.sync_copy(zer.at[pl.ds(0, 1808)], acc.at[pl.ds(8192, 1808)])

    plsc.subcore_barrier()

    ebase = wid * EPW

    def chunk_body(k, carry):
        off = ebase + k * CHUNK
        pltpu.sync_copy(dst_h.at[pl.ds(off, CHUNK)], dstv)
        pltpu.sync_copy(ew_h.at[pl.ds(off, CHUNK)], ewv)
        pltpu.sync_copy(ewv, acc.at[dstv], add=True)
        return carry

    lax.fori_loop(0, NCHUNK, chunk_body, 0)
    plsc.subcore_barrier()

    @pl.when(s < 4)
    def _():
        pltpu.sync_copy(acc.at[pl.ds(s * 2048, 2048)], zer)
        pltpu.sync_copy(zer, out_h.at[pl.ds(c * N_NODES + s * 2048, 2048)])

    @pl.when(s == 4)
    def _():
        pltpu.sync_copy(acc.at[pl.ds(8192, 1808)], zer.at[pl.ds(0, 1808)])
        pltpu.sync_copy(
            zer.at[pl.ds(0, 1808)], out_h.at[pl.ds(c * N_NODES + 8192, 1808)]
        )


_deg_call = functools.partial(
    pl.kernel,
    out_type=jax.ShapeDtypeStruct((2 * N_NODES,), jnp.float32),
    mesh=_MESH,
    scratch_types=[
        pltpu.VMEM((CHUNK,), jnp.int32),
        pltpu.VMEM((CHUNK,), jnp.float32),
        pltpu.VMEM((2048,), jnp.float32),
        pltpu.VMEM_SHARED((N_NODES,), jnp.float32),
    ],
)(_deg_body)


def _spmm_body(src_h, dst_h, ew_h, xs_h, out_h, srcv, dstv, ewv, rows, sem, acc):
    c = lax.axis_index("c")
    s = lax.axis_index("s")
    wid = s * NC + c

    def zrow(e, carry):
        for j in range(8):
            rows[e, pl.ds(j * 16, 16)] = jnp.zeros((16,), jnp.float32)
        return carry

    lax.fori_loop(0, 16, zrow, 0)
    base_r = s * 624
    nblk = 39 + jnp.where(s == 15, 1, 0)

    def zcopy(q, carry):
        pltpu.sync_copy(
            rows.at[pl.ds(0, 16)], acc.at[pl.ds(base_r + q * 16, 16)]
        )
        return carry

    lax.fori_loop(0, nblk, zcopy, 0)
    plsc.subcore_barrier()

    ebase = wid * EPW

    def chunk_body(k, carry):
        off = ebase + k * CHUNK
        pltpu.sync_copy(src_h.at[pl.ds(off, CHUNK)], srcv)
        pltpu.sync_copy(dst_h.at[pl.ds(off, CHUNK)], dstv)
        pltpu.sync_copy(ew_h.at[pl.ds(off, CHUNK)], ewv)
        pltpu.async_copy(xs_h.at[srcv], rows, sem).wait()

        def group_body(g, inner):
            gbase = pl.multiple_of(g * 16, 16)
            wvec = ewv[pl.ds(gbase, 16)]
            for i in range(16):
                w = wvec[i]
                e = gbase + i
                for j in range(8):
                    rows[e, pl.ds(j * 16, 16)] = rows[e, pl.ds(j * 16, 16)] * w
            return inner

        lax.fori_loop(0, CHUNK // 16, group_body, 0)
        pltpu.sync_copy(rows, acc.at[dstv], add=True)
        return carry

    lax.fori_loop(0, NCHUNK, chunk_body, 0)
    plsc.subcore_barrier()

    out_base = c * N_NODES + s * 624
    pltpu.sync_copy(acc.at[pl.ds(s * 624, 624)], out_h.at[pl.ds(out_base, 624)])

    @pl.when(s == 15)
    def _():
        pltpu.sync_copy(
            acc.at[pl.ds(9984, 16)], out_h.at[pl.ds(c * N_NODES + 9984, 16)]
        )


_spmm_call = functools.partial(
    pl.kernel,
    out_type=jax.ShapeDtypeStruct((2 * N_NODES, D_AGG), jnp.float32),
    mesh=_MESH,
    scratch_types=[
        pltpu.VMEM((CHUNK,), jnp.int32),
        pltpu.VMEM((CHUNK,), jnp.int32),
        pltpu.VMEM((CHUNK,), jnp.float32),
        pltpu.VMEM((CHUNK, D_AGG), jnp.float32),
        pltpu.SemaphoreType.DMA,
        pltpu.VMEM_SHARED((N_NODES, D_AGG), jnp.float32),
    ],
)(_spmm_body)


_BLK = 1000
_GRID = N_NODES // _BLK


def _dinv_block(p0, p1):
    deg = p0 + p1 + 1.0
    return jnp.where(deg > 0, lax.rsqrt(deg), 0.0)


def _prep_body(p0_ref, p1_ref, x_ref, xs_ref):
    dinv = _dinv_block(p0_ref[...], p1_ref[...])
    xs_ref[...] = x_ref[...] * dinv


def _layer1_body(z0_ref, z1_ref, xs_ref, p0_ref, p1_ref, w1_ref, b1_ref,
                 w2_ref, out_ref):
    dinv = _dinv_block(p0_ref[...], p1_ref[...])
    y1 = (z0_ref[...] + z1_ref[...] + xs_ref[...]) * dinv
    h1 = jnp.maximum(
        jnp.dot(y1, w1_ref[...], preferred_element_type=jnp.float32)
        + b1_ref[...],
        0.0,
    )
    out_ref[...] = (
        jnp.dot(h1, w2_ref[...], preferred_element_type=jnp.float32) * dinv
    )


def _layer2_body(z0_ref, z1_ref, xs_ref, p0_ref, p1_ref, b2_ref, wfc_ref,
                 bfc_ref, out_ref):
    dinv = _dinv_block(p0_ref[...], p1_ref[...])
    y2 = (z0_ref[...] + z1_ref[...] + xs_ref[...]) * dinv
    h2 = jnp.maximum(y2 + b2_ref[...], 0.0)
    out_ref[...] = (
        jnp.dot(h2, wfc_ref[...], preferred_element_type=jnp.float32)
        + bfc_ref[...]
    )


def _row_spec(width):
    return pl.BlockSpec((_BLK, width), lambda i: (i, 0))


def _full_spec(shape):
    return pl.BlockSpec(shape, lambda i: (0, 0))


_prep_call = pl.pallas_call(
    _prep_body,
    grid=(_GRID,),
    in_specs=[_row_spec(1), _row_spec(1), _row_spec(D_AGG)],
    out_specs=_row_spec(D_AGG),
    out_shape=jax.ShapeDtypeStruct((N_NODES, D_AGG), jnp.float32),
)

_layer1_call = pl.pallas_call(
    _layer1_body,
    grid=(_GRID,),
    in_specs=[
        _row_spec(D_AGG), _row_spec(D_AGG), _row_spec(D_AGG),
        _row_spec(1), _row_spec(1),
        _full_spec((128, 256)), _full_spec((1, 256)), _full_spec((256, 128)),
    ],
    out_specs=_row_spec(D_AGG),
    out_shape=jax.ShapeDtypeStruct((N_NODES, D_AGG), jnp.float32),
)

_layer2_call = pl.pallas_call(
    _layer2_body,
    grid=(_GRID,),
    in_specs=[
        _row_spec(D_AGG), _row_spec(D_AGG), _row_spec(D_AGG),
        _row_spec(1), _row_spec(1),
        _full_spec((1, 128)), _full_spec((128, 1)), _full_spec((1, 1)),
    ],
    out_specs=_row_spec(1),
    out_shape=jax.ShapeDtypeStruct((N_NODES, 1), jnp.float32),
)


def kernel(x, edge_index, edge_attr, W1, b1, W2, b2, Wfc, bfc):
    src = edge_index[0]
    dst = edge_index[1]
    ew = edge_attr

    degp = _deg_call(dst, ew)
    p0 = degp[:N_NODES].reshape(N_NODES, 1)
    p1 = degp[N_NODES:].reshape(N_NODES, 1)

    xs1 = _prep_call(p0, p1, x)
    z = _spmm_call(src, dst, ew, xs1)
    xs2 = _layer1_call(
        z[:N_NODES], z[N_NODES:], xs1, p0, p1, W1, b1.reshape(1, -1), W2
    )
    z2 = _spmm_call(src, dst, ew, xs2)
    out = _layer2_call(
        z2[:N_NODES], z2[N_NODES:], xs2, p0, p1,
        b2.reshape(1, -1), Wfc, bfc.reshape(1, 1),
    )
    return out


# speedup vs baseline: 11.4808x; 1.0724x over previous
import functools

import jax
import jax.numpy as jnp
from jax import lax
from jax.experimental import pallas as pl
from jax.experimental.pallas import tpu as pltpu
from jax.experimental.pallas import tpu_sc as plsc

N_NODES = 10000
D_AGG = 128
E_TOTAL = 320000

NC, NS = 2, 16
NW = NC * NS
EPW = E_TOTAL // NW
CHUNK = 80
NCHUNK = EPW // CHUNK
ROWS_PT = N_NODES // NS

_MESH = plsc.VectorSubcoreMesh(core_axis_name="c", subcore_axis_name="s")


def _deg_body(dst_h, ew_h, out_h, dstv, ewv, zer, dsem, acc):
    c = lax.axis_index("c")
    s = lax.axis_index("s")
    wid = s * NC + c

    pltpu.sync_copy(dst_h.at[wid], dstv)
    pltpu.sync_copy(ew_h.at[wid], ewv)

    def zinit(i, carry):
        zer[pl.ds(pl.multiple_of(i * 16, 16), 16)] = jnp.zeros((16,), jnp.float32)
        return carry

    lax.fori_loop(0, 128, zinit, 0)

    @pl.when(s < 4)
    def _():
        pltpu.sync_copy(zer, acc.at[pl.ds(s * 2048, 2048)])

    @pl.when(s == 4)
    def _():
        pltpu.sync_copy(zer.at[pl.ds(0, 1808)], acc.at[pl.ds(8192, 1808)])

    plsc.subcore_barrier()

    def chunk_body(k, carry):
        pltpu.sync_copy(ewv.at[k], acc.at[dstv.at[k]], add=True)
        return carry

    lax.fori_loop(0, NCHUNK, chunk_body, 0)
    plsc.subcore_barrier()

    @pl.when(s < 4)
    def _():
        pltpu.sync_copy(acc.at[pl.ds(s * 2048, 2048)], zer)
        pltpu.sync_copy(zer, out_h.at[pl.ds(c * N_NODES + s * 2048, 2048)])

    @pl.when(s == 4)
    def _():
        pltpu.sync_copy(acc.at[pl.ds(8192, 1808)], zer.at[pl.ds(0, 1808)])
        pltpu.sync_copy(
            zer.at[pl.ds(0, 1808)], out_h.at[pl.ds(c * N_NODES + 8192, 1808)]
        )


_deg_call = functools.partial(
    pl.kernel,
    out_type=jax.ShapeDtypeStruct((2 * N_NODES,), jnp.float32),
    mesh=_MESH,
    scratch_types=[
        pltpu.VMEM((NCHUNK, CHUNK), jnp.int32),
        pltpu.VMEM((NCHUNK, CHUNK), jnp.float32),
        pltpu.VMEM((2048,), jnp.float32),
        pltpu.SemaphoreType.DMA,
        pltpu.VMEM_SHARED((N_NODES,), jnp.float32),
    ],
)(_deg_body)


def _spmm_body(src_h, dst_h, ew_h, xs_h, out_h, srcv, dstv, ewv,
               rows0, rows1, gsem0, gsem1, ssem0, ssem1, isem0, isem1, acc):
    c = lax.axis_index("c")
    s = lax.axis_index("s")
    wid = s * NC + c

    def zrow(e, carry):
        for j in range(8):
            rows0[e, pl.ds(j * 16, 16)] = jnp.zeros((16,), jnp.float32)
        return carry

    lax.fori_loop(0, 16, zrow, 0)
    base_r = s * 624
    nblk = 39 + jnp.where(s == 15, 1, 0)

    def zcopy(q, carry):
        pltpu.sync_copy(
            rows0.at[pl.ds(0, 16)], acc.at[pl.ds(base_r + q * 16, 16)]
        )
        return carry

    lax.fori_loop(0, nblk, zcopy, 0)
    plsc.subcore_barrier()

    def idx_load(k, j, sem):
        pltpu.async_copy(src_h.at[wid, k], srcv.at[j], sem)
        pltpu.async_copy(dst_h.at[wid, k], dstv.at[j], sem)
        pltpu.async_copy(ew_h.at[wid, k], ewv.at[j], sem)

    def idx_drain(j, sem):
        pltpu.make_async_copy(src_h.at[0, 0], srcv.at[j], sem).wait()
        pltpu.make_async_copy(dst_h.at[0, 0], dstv.at[j], sem).wait()
        pltpu.make_async_copy(ew_h.at[0, 0], ewv.at[j], sem).wait()

    def gather(j, rows, sem):
        pltpu.async_copy(xs_h.at[srcv.at[j]], rows, sem)

    def scatter(j, rows, sem):
        pltpu.async_copy(rows, acc.at[dstv.at[j]], sem, add=True)

    def drain(rows, sem):
        pltpu.make_async_copy(xs_h.at[pl.ds(0, CHUNK)], rows, sem).wait()

    def scale(rows, j):
        def group_body(g, inner):
            gbase = pl.multiple_of(g * 16, 16)
            wvec = ewv[j, pl.ds(gbase, 16)]
            for i in range(16):
                w = wvec[i]
                e = gbase + i
                for jj in range(8):
                    rows[e, pl.ds(jj * 16, 16)] = (
                        rows[e, pl.ds(jj * 16, 16)] * w
                    )
            return inner

        lax.fori_loop(0, CHUNK // 16, group_body, 0)

    def chunk_body(k, carry):
        pltpu.sync_copy(src_h.at[wid, k], srcv.at[0])
        pltpu.sync_copy(dst_h.at[wid, k], dstv.at[0])
        pltpu.sync_copy(ew_h.at[wid, k], ewv.at[0])
        pltpu.async_copy(xs_h.at[srcv.at[0]], rows0, gsem0).wait()
        scale(rows0, 0)
        pltpu.sync_copy(rows0, acc.at[dstv.at[0]], add=True)
        return carry

    lax.fori_loop(0, NCHUNK, chunk_body, 0)
    plsc.subcore_barrier()

    out_base = c * N_NODES + s * 624
    pltpu.sync_copy(acc.at[pl.ds(s * 624, 624)], out_h.at[pl.ds(out_base, 624)])

    @pl.when(s == 15)
    def _():
        pltpu.sync_copy(
            acc.at[pl.ds(9984, 16)], out_h.at[pl.ds(c * N_NODES + 9984, 16)]
        )


_spmm_call = functools.partial(
    pl.kernel,
    out_type=jax.ShapeDtypeStruct((2 * N_NODES, D_AGG), jnp.float32),
    mesh=_MESH,
    scratch_types=[
        pltpu.VMEM((2, CHUNK), jnp.int32),
        pltpu.VMEM((2, CHUNK), jnp.int32),
        pltpu.VMEM((2, CHUNK), jnp.float32),
        pltpu.VMEM((CHUNK, D_AGG), jnp.float32),
        pltpu.VMEM((CHUNK, D_AGG), jnp.float32),
        pltpu.SemaphoreType.DMA,
        pltpu.SemaphoreType.DMA,
        pltpu.SemaphoreType.DMA,
        pltpu.SemaphoreType.DMA,
        pltpu.SemaphoreType.DMA,
        pltpu.SemaphoreType.DMA,
        pltpu.VMEM_SHARED((N_NODES, D_AGG), jnp.float32),
    ],
)(_spmm_body)


_BLK = 1000
_GRID = N_NODES // _BLK


def _dinv_block(p0, p1):
    deg = p0 + p1 + 1.0
    return jnp.where(deg > 0, lax.rsqrt(deg), 0.0)


def _prep_body(p0_ref, p1_ref, x_ref, xs_ref):
    dinv = _dinv_block(p0_ref[...], p1_ref[...])
    xs_ref[...] = x_ref[...] * dinv


def _layer1_body(z0_ref, z1_ref, xs_ref, p0_ref, p1_ref, w1_ref, b1_ref,
                 w2_ref, out_ref):
    dinv = _dinv_block(p0_ref[...], p1_ref[...])
    y1 = (z0_ref[...] + z1_ref[...] + xs_ref[...]) * dinv
    h1 = jnp.maximum(
        jnp.dot(y1, w1_ref[...], preferred_element_type=jnp.float32)
        + b1_ref[...],
        0.0,
    )
    out_ref[...] = (
        jnp.dot(h1, w2_ref[...], preferred_element_type=jnp.float32) * dinv
    )


def _layer2_body(z0_ref, z1_ref, xs_ref, p0_ref, p1_ref, b2_ref, wfc_ref,
                 bfc_ref, out_ref):
    dinv = _dinv_block(p0_ref[...], p1_ref[...])
    y2 = (z0_ref[...] + z1_ref[...] + xs_ref[...]) * dinv
    h2 = jnp.maximum(y2 + b2_ref[...], 0.0)
    out_ref[...] = (
        jnp.dot(h2, wfc_ref[...], preferred_element_type=jnp.float32)
        + bfc_ref[...]
    )


def _row_spec(width):
    return pl.BlockSpec((_BLK, width), lambda i: (i, 0))


def _full_spec(shape):
    return pl.BlockSpec(shape, lambda i: (0, 0))


_prep_call = pl.pallas_call(
    _prep_body,
    grid=(_GRID,),
    in_specs=[_row_spec(1), _row_spec(1), _row_spec(D_AGG)],
    out_specs=_row_spec(D_AGG),
    out_shape=jax.ShapeDtypeStruct((N_NODES, D_AGG), jnp.float32),
)

_layer1_call = pl.pallas_call(
    _layer1_body,
    grid=(_GRID,),
    in_specs=[
        _row_spec(D_AGG), _row_spec(D_AGG), _row_spec(D_AGG),
        _row_spec(1), _row_spec(1),
        _full_spec((128, 256)), _full_spec((1, 256)), _full_spec((256, 128)),
    ],
    out_specs=_row_spec(D_AGG),
    out_shape=jax.ShapeDtypeStruct((N_NODES, D_AGG), jnp.float32),
)

_layer2_call = pl.pallas_call(
    _layer2_body,
    grid=(_GRID,),
    in_specs=[
        _row_spec(D_AGG), _row_spec(D_AGG), _row_spec(D_AGG),
        _row_spec(1), _row_spec(1),
        _full_spec((1, 128)), _full_spec((128, 1)), _full_spec((1, 1)),
    ],
    out_specs=_row_spec(1),
    out_shape=jax.ShapeDtypeStruct((N_NODES, 1), jnp.float32),
)


def kernel(x, edge_index, edge_attr, W1, b1, W2, b2, Wfc, bfc):
    src = edge_index[0].reshape(NW, NCHUNK, CHUNK)
    dst = edge_index[1].reshape(NW, NCHUNK, CHUNK)
    ew = edge_attr.reshape(NW, NCHUNK, CHUNK)

    degp = _deg_call(dst, ew)
    p0 = degp[:N_NODES].reshape(N_NODES, 1)
    p1 = degp[N_NODES:].reshape(N_NODES, 1)

    xs1 = _prep_call(p0, p1, x)
    z = _spmm_call(src, dst, ew, xs1)
    xs2 = _layer1_call(
        z[:N_NODES], z[N_NODES:], xs1, p0, p1, W1, b1.reshape(1, -1), W2
    )
    z2 = _spmm_call(src, dst, ew, xs2)
    out = _layer2_call(
        z2[:N_NODES], z2[N_NODES:], xs2, p0, p1,
        b2.reshape(1, -1), Wfc, bfc.reshape(1, 1),
    )
    return out


# speedup vs baseline: 18.6416x; 1.6237x over previous
import functools

import jax
import jax.numpy as jnp
from jax import lax
from jax.experimental import pallas as pl
from jax.experimental.pallas import tpu as pltpu
from jax.experimental.pallas import tpu_sc as plsc

N_NODES = 10000
D_AGG = 128
E_TOTAL = 320000

NC, NS = 2, 16
NW = NC * NS
EPW = E_TOTAL // NW
CHUNK = 80
NCHUNK = EPW // CHUNK
ROWS_PT = N_NODES // NS

_MESH = plsc.VectorSubcoreMesh(core_axis_name="c", subcore_axis_name="s")


def _deg_body(dst_h, ew_h, out_h, dstv, ewv, zer, dsem, acc):
    c = lax.axis_index("c")
    s = lax.axis_index("s")
    wid = s * NC + c

    pltpu.sync_copy(dst_h.at[wid], dstv)
    pltpu.sync_copy(ew_h.at[wid], ewv)

    def zinit(i, carry):
        zer[pl.ds(pl.multiple_of(i * 16, 16), 16)] = jnp.zeros((16,), jnp.float32)
        return carry

    lax.fori_loop(0, 128, zinit, 0)

    @pl.when(s < 4)
    def _():
        pltpu.sync_copy(zer, acc.at[pl.ds(s * 2048, 2048)])

    @pl.when(s == 4)
    def _():
        pltpu.sync_copy(zer.at[pl.ds(0, 1808)], acc.at[pl.ds(8192, 1808)])

    plsc.subcore_barrier()

    def chunk_body(k, carry):
        pltpu.sync_copy(ewv.at[k], acc.at[dstv.at[k]], add=True)
        return carry

    lax.fori_loop(0, NCHUNK, chunk_body, 0)
    plsc.subcore_barrier()

    @pl.when(s < 4)
    def _():
        pltpu.sync_copy(acc.at[pl.ds(s * 2048, 2048)], zer)
        pltpu.sync_copy(zer, out_h.at[pl.ds(c * N_NODES + s * 2048, 2048)])

    @pl.when(s == 4)
    def _():
        pltpu.sync_copy(acc.at[pl.ds(8192, 1808)], zer.at[pl.ds(0, 1808)])
        pltpu.sync_copy(
            zer.at[pl.ds(0, 1808)], out_h.at[pl.ds(c * N_NODES + 8192, 1808)]
        )


_deg_call = functools.partial(
    pl.kernel,
    out_type=jax.ShapeDtypeStruct((2 * N_NODES,), jnp.float32),
    mesh=_MESH,
    scratch_types=[
        pltpu.VMEM((NCHUNK, CHUNK), jnp.int32),
        pltpu.VMEM((NCHUNK, CHUNK), jnp.float32),
        pltpu.VMEM((2048,), jnp.float32),
        pltpu.SemaphoreType.DMA,
        pltpu.VMEM_SHARED((N_NODES,), jnp.float32),
    ],
)(_deg_body)


def _spmm_body(src_h, dst_h, ew_h, xs_h, out_h, srcv, dstv, ewv,
               rows0, rows1, gsem0, gsem1, ssem0, ssem1, isem0, isem1, acc):
    c = lax.axis_index("c")
    s = lax.axis_index("s")
    wid = s * NC + c

    def zrow(e, carry):
        for j in range(8):
            rows0[e, pl.ds(j * 16, 16)] = jnp.zeros((16,), jnp.float32)
        return carry

    lax.fori_loop(0, 16, zrow, 0)
    base_r = s * 624
    nblk = 39 + jnp.where(s == 15, 1, 0)

    def zcopy(q, carry):
        pltpu.sync_copy(
            rows0.at[pl.ds(0, 16)], acc.at[pl.ds(base_r + q * 16, 16)]
        )
        return carry

    lax.fori_loop(0, nblk, zcopy, 0)
    plsc.subcore_barrier()

    def idx_load(k, j, sem):
        pltpu.async_copy(src_h.at[wid, k], srcv.at[j], sem)
        pltpu.async_copy(dst_h.at[wid, k], dstv.at[j], sem)
        pltpu.async_copy(ew_h.at[wid, k], ewv.at[j], sem)

    def idx_drain(j, sem):
        pltpu.make_async_copy(src_h.at[0, 0], srcv.at[j], sem).wait()
        pltpu.make_async_copy(dst_h.at[0, 0], dstv.at[j], sem).wait()
        pltpu.make_async_copy(ew_h.at[0, 0], ewv.at[j], sem).wait()

    def gather(j, rows, sem):
        pltpu.async_copy(xs_h.at[srcv.at[j]], rows, sem)

    def scatter(j, rows, sem):
        pltpu.async_copy(rows, acc.at[dstv.at[j]], sem, add=True)

    def drain(rows, sem):
        pltpu.make_async_copy(xs_h.at[pl.ds(0, CHUNK)], rows, sem).wait()

    def scale(rows, j):
        def group_body(g, inner):
            gbase = pl.multiple_of(g * 16, 16)
            wvec = ewv[j, pl.ds(gbase, 16)]
            for i in range(16):
                w = wvec[i]
                e = gbase + i
                for jj in range(8):
                    rows[e, pl.ds(jj * 16, 16)] = (
                        rows[e, pl.ds(jj * 16, 16)] * w
                    )
            return inner

        lax.fori_loop(0, CHUNK // 16, group_body, 0)

    idx_load(0, 0, isem0)
    idx_load(1, 1, isem1)

    def pair(t, carry):
        a = 2 * t
        idx_drain(0, isem0)
        idx_drain(1, isem1)
        d0 = pltpu.async_copy(xs_h.at[srcv.at[0]], rows0, gsem0)
        d1 = pltpu.async_copy(xs_h.at[srcv.at[1]], rows1, gsem1)
        d0.wait()
        scale(rows0, 0)
        pltpu.sync_copy(rows0, acc.at[dstv.at[0]], add=True)
        d1.wait()
        scale(rows1, 1)
        idx_load(a + 2, 0, isem0)
        pltpu.sync_copy(rows1, acc.at[dstv.at[1]], add=True)

        @pl.when(a + 3 < NCHUNK)
        def _():
            idx_load(a + 3, 1, isem1)

        return carry

    lax.fori_loop(0, NCHUNK // 2, pair, 0)
    idx_drain(0, isem0)
    pltpu.async_copy(xs_h.at[srcv.at[0]], rows0, gsem0).wait()
    scale(rows0, 0)
    pltpu.sync_copy(rows0, acc.at[dstv.at[0]], add=True)
    plsc.subcore_barrier()

    out_base = c * N_NODES + s * 624
    pltpu.sync_copy(acc.at[pl.ds(s * 624, 624)], out_h.at[pl.ds(out_base, 624)])

    @pl.when(s == 15)
    def _():
        pltpu.sync_copy(
            acc.at[pl.ds(9984, 16)], out_h.at[pl.ds(c * N_NODES + 9984, 16)]
        )


_spmm_call = functools.partial(
    pl.kernel,
    out_type=jax.ShapeDtypeStruct((2 * N_NODES, D_AGG), jnp.float32),
    mesh=_MESH,
    scratch_types=[
        pltpu.VMEM((2, CHUNK), jnp.int32),
        pltpu.VMEM((2, CHUNK), jnp.int32),
        pltpu.VMEM((2, CHUNK), jnp.float32),
        pltpu.VMEM((CHUNK, D_AGG), jnp.float32),
        pltpu.VMEM((CHUNK, D_AGG), jnp.float32),
        pltpu.SemaphoreType.DMA,
        pltpu.SemaphoreType.DMA,
        pltpu.SemaphoreType.DMA,
        pltpu.SemaphoreType.DMA,
        pltpu.SemaphoreType.DMA,
        pltpu.SemaphoreType.DMA,
        pltpu.VMEM_SHARED((N_NODES, D_AGG), jnp.float32),
    ],
)(_spmm_body)


_BLK = 1000
_GRID = N_NODES // _BLK


def _dinv_block(p0, p1):
    deg = p0 + p1 + 1.0
    return jnp.where(deg > 0, lax.rsqrt(deg), 0.0)


def _prep_body(p0_ref, p1_ref, x_ref, xs_ref):
    dinv = _dinv_block(p0_ref[...], p1_ref[...])
    xs_ref[...] = x_ref[...] * dinv


def _layer1_body(z0_ref, z1_ref, xs_ref, p0_ref, p1_ref, w1_ref, b1_ref,
                 w2_ref, out_ref):
    dinv = _dinv_block(p0_ref[...], p1_ref[...])
    y1 = (z0_ref[...] + z1_ref[...] + xs_ref[...]) * dinv
    h1 = jnp.maximum(
        jnp.dot(y1, w1_ref[...], preferred_element_type=jnp.float32)
        + b1_ref[...],
        0.0,
    )
    out_ref[...] = (
        jnp.dot(h1, w2_ref[...], preferred_element_type=jnp.float32) * dinv
    )


def _layer2_body(z0_ref, z1_ref, xs_ref, p0_ref, p1_ref, b2_ref, wfc_ref,
                 bfc_ref, out_ref):
    dinv = _dinv_block(p0_ref[...], p1_ref[...])
    y2 = (z0_ref[...] + z1_ref[...] + xs_ref[...]) * dinv
    h2 = jnp.maximum(y2 + b2_ref[...], 0.0)
    out_ref[...] = (
        jnp.dot(h2, wfc_ref[...], preferred_element_type=jnp.float32)
        + bfc_ref[...]
    )


def _row_spec(width):
    return pl.BlockSpec((_BLK, width), lambda i: (i, 0))


def _full_spec(shape):
    return pl.BlockSpec(shape, lambda i: (0, 0))


_prep_call = pl.pallas_call(
    _prep_body,
    grid=(_GRID,),
    in_specs=[_row_spec(1), _row_spec(1), _row_spec(D_AGG)],
    out_specs=_row_spec(D_AGG),
    out_shape=jax.ShapeDtypeStruct((N_NODES, D_AGG), jnp.float32),
)

_layer1_call = pl.pallas_call(
    _layer1_body,
    grid=(_GRID,),
    in_specs=[
        _row_spec(D_AGG), _row_spec(D_AGG), _row_spec(D_AGG),
        _row_spec(1), _row_spec(1),
        _full_spec((128, 256)), _full_spec((1, 256)), _full_spec((256, 128)),
    ],
    out_specs=_row_spec(D_AGG),
    out_shape=jax.ShapeDtypeStruct((N_NODES, D_AGG), jnp.float32),
)

_layer2_call = pl.pallas_call(
    _layer2_body,
    grid=(_GRID,),
    in_specs=[
        _row_spec(D_AGG), _row_spec(D_AGG), _row_spec(D_AGG),
        _row_spec(1), _row_spec(1),
        _full_spec((1, 128)), _full_spec((128, 1)), _full_spec((1, 1)),
    ],
    out_specs=_row_spec(1),
    out_shape=jax.ShapeDtypeStruct((N_NODES, 1), jnp.float32),
)


def kernel(x, edge_index, edge_attr, W1, b1, W2, b2, Wfc, bfc):
    src = edge_index[0].reshape(NW, NCHUNK, CHUNK)
    dst = edge_index[1].reshape(NW, NCHUNK, CHUNK)
    ew = edge_attr.reshape(NW, NCHUNK, CHUNK)

    degp = _deg_call(dst, ew)
    p0 = degp[:N_NODES].reshape(N_NODES, 1)
    p1 = degp[N_NODES:].reshape(N_NODES, 1)

    xs1 = _prep_call(p0, p1, x)
    z = _spmm_call(src, dst, ew, xs1)
    xs2 = _layer1_call(
        z[:N_NODES], z[N_NODES:], xs1, p0, p1, W1, b1.reshape(1, -1), W2
    )
    z2 = _spmm_call(src, dst, ew, xs2)
    out = _layer2_call(
        z2[:N_NODES], z2[N_NODES:], xs2, p0, p1,
        b2.reshape(1, -1), Wfc, bfc.reshape(1, 1),
    )
    return out


# speedup vs baseline: 23.7452x; 1.2738x over previous
import functools

import jax
import jax.numpy as jnp
from jax import lax
from jax.experimental import pallas as pl
from jax.experimental.pallas import tpu as pltpu
from jax.experimental.pallas import tpu_sc as plsc

N_NODES = 10000
D_AGG = 128
E_TOTAL = 320000

NC, NS = 2, 16
NW = NC * NS
EPW = E_TOTAL // NW
CHUNK = 80
NCHUNK = EPW // CHUNK
ROWS_PT = N_NODES // NS

_MESH = plsc.VectorSubcoreMesh(core_axis_name="c", subcore_axis_name="s")


def _deg_body(dst_h, ew_h, out_h, dstv, ewv, zer, dsem, acc):
    c = lax.axis_index("c")
    s = lax.axis_index("s")
    wid = s * NC + c

    pltpu.sync_copy(dst_h.at[wid], dstv)
    pltpu.sync_copy(ew_h.at[wid], ewv)

    def zinit(i, carry):
        zer[pl.ds(pl.multiple_of(i * 16, 16), 16)] = jnp.zeros((16,), jnp.float32)
        return carry

    lax.fori_loop(0, 128, zinit, 0)

    @pl.when(s < 4)
    def _():
        pltpu.sync_copy(zer, acc.at[pl.ds(s * 2048, 2048)])

    @pl.when(s == 4)
    def _():
        pltpu.sync_copy(zer.at[pl.ds(0, 1808)], acc.at[pl.ds(8192, 1808)])

    plsc.subcore_barrier()

    def chunk_body(k, carry):
        pltpu.sync_copy(ewv.at[k], acc.at[dstv.at[k]], add=True)
        return carry

    lax.fori_loop(0, NCHUNK, chunk_body, 0)
    plsc.subcore_barrier()

    @pl.when(s < 4)
    def _():
        pltpu.sync_copy(acc.at[pl.ds(s * 2048, 2048)], zer)
        pltpu.sync_copy(zer, out_h.at[pl.ds(c * N_NODES + s * 2048, 2048)])

    @pl.when(s == 4)
    def _():
        pltpu.sync_copy(acc.at[pl.ds(8192, 1808)], zer.at[pl.ds(0, 1808)])
        pltpu.sync_copy(
            zer.at[pl.ds(0, 1808)], out_h.at[pl.ds(c * N_NODES + 8192, 1808)]
        )


_deg_call = functools.partial(
    pl.kernel,
    out_type=jax.ShapeDtypeStruct((2 * N_NODES,), jnp.float32),
    mesh=_MESH,
    scratch_types=[
        pltpu.VMEM((NCHUNK, CHUNK), jnp.int32),
        pltpu.VMEM((NCHUNK, CHUNK), jnp.float32),
        pltpu.VMEM((2048,), jnp.float32),
        pltpu.SemaphoreType.DMA,
        pltpu.VMEM_SHARED((N_NODES,), jnp.float32),
    ],
)(_deg_body)


def _spmm_body(src_h, dst_h, ew_h, xs_h, out_h, srcv, dstv, ewv,
               rows0, rows1, gsem0, gsem1, ssem0, ssem1,
               isem0, isem1, isem2, isem3, acc):
    c = lax.axis_index("c")
    s = lax.axis_index("s")
    wid = s * NC + c

    def zrow(e, carry):
        for j in range(8):
            rows0[e, pl.ds(j * 16, 16)] = jnp.zeros((16,), jnp.float32)
        return carry

    lax.fori_loop(0, 16, zrow, 0)
    base_r = s * 624
    nblk = 39 + jnp.where(s == 15, 1, 0)

    def zcopy(q, carry):
        pltpu.sync_copy(
            rows0.at[pl.ds(0, 16)], acc.at[pl.ds(base_r + q * 16, 16)]
        )
        return carry

    lax.fori_loop(0, nblk, zcopy, 0)
    plsc.subcore_barrier()

    def idx_load(k, j, sem):
        pltpu.async_copy(src_h.at[wid, k], srcv.at[j], sem)
        pltpu.async_copy(dst_h.at[wid, k], dstv.at[j], sem)
        pltpu.async_copy(ew_h.at[wid, k], ewv.at[j], sem)

    def idx_drain(j, sem):
        pltpu.make_async_copy(src_h.at[0, 0], srcv.at[j], sem).wait()
        pltpu.make_async_copy(dst_h.at[0, 0], dstv.at[j], sem).wait()
        pltpu.make_async_copy(ew_h.at[0, 0], ewv.at[j], sem).wait()

    def gather(j, rows, sem):
        return pltpu.async_copy(xs_h.at[srcv.at[j]], rows, sem)

    def scatter(j, rows, sem):
        return pltpu.async_copy(rows, acc.at[dstv.at[j]], sem, add=True)

    def drain(rows, sem):
        pltpu.make_async_copy(xs_h.at[pl.ds(0, CHUNK)], rows, sem).wait()

    def scale(rows, j):
        def group_body(g, inner):
            gbase = pl.multiple_of(g * 16, 16)
            wvec = ewv[j, pl.ds(gbase, 16)]
            for i in range(16):
                w = wvec[i]
                e = gbase + i
                for jj in range(8):
                    rows[e, pl.ds(jj * 16, 16)] = (
                        rows[e, pl.ds(jj * 16, 16)] * w
                    )
            return inner

        lax.fori_loop(0, CHUNK // 16, group_body, 0)

    idx_load(0, 0, isem0)
    idx_load(1, 1, isem1)

    def quad(q, carry):
        a = 4 * q
        idx_drain(0, isem0)
        g0 = gather(0, rows0, gsem0)
        idx_drain(1, isem1)
        g1 = gather(1, rows1, gsem1)
        idx_load(a + 2, 2, isem2)
        idx_load(a + 3, 3, isem3)
        g0.wait()
        scale(rows0, 0)
        s0 = scatter(0, rows0, ssem0)
        g1.wait()
        scale(rows1, 1)
        s1 = scatter(1, rows1, ssem1)
        s0.wait()
        idx_drain(2, isem2)
        g0 = gather(2, rows0, gsem0)
        s1.wait()
        idx_load(a + 4, 0, isem0)

        @pl.when(a + 5 < NCHUNK)
        def _():
            idx_load(a + 5, 1, isem1)

        idx_drain(3, isem3)
        g1 = gather(3, rows1, gsem1)
        g0.wait()
        scale(rows0, 2)
        s0 = scatter(2, rows0, ssem0)
        g1.wait()
        scale(rows1, 3)
        s1 = scatter(3, rows1, ssem1)
        s0.wait()
        s1.wait()
        return carry

    lax.fori_loop(0, (NCHUNK - 1) // 4, quad, 0)
    idx_drain(0, isem0)
    gather(0, rows0, gsem0).wait()
    scale(rows0, 0)
    pltpu.sync_copy(rows0, acc.at[dstv.at[0]], add=True)
    plsc.subcore_barrier()

    out_base = c * N_NODES + s * 624
    pltpu.sync_copy(acc.at[pl.ds(s * 624, 624)], out_h.at[pl.ds(out_base, 624)])

    @pl.when(s == 15)
    def _():
        pltpu.sync_copy(
            acc.at[pl.ds(9984, 16)], out_h.at[pl.ds(c * N_NODES + 9984, 16)]
        )


_spmm_call = functools.partial(
    pl.kernel,
    out_type=jax.ShapeDtypeStruct((2 * N_NODES, D_AGG), jnp.float32),
    mesh=_MESH,
    scratch_types=[
        pltpu.VMEM((4, CHUNK), jnp.int32),
        pltpu.VMEM((4, CHUNK), jnp.int32),
        pltpu.VMEM((4, CHUNK), jnp.float32),
        pltpu.VMEM((CHUNK, D_AGG), jnp.float32),
        pltpu.VMEM((CHUNK, D_AGG), jnp.float32),
        pltpu.SemaphoreType.DMA,
        pltpu.SemaphoreType.DMA,
        pltpu.SemaphoreType.DMA,
        pltpu.SemaphoreType.DMA,
        pltpu.SemaphoreType.DMA,
        pltpu.SemaphoreType.DMA,
        pltpu.SemaphoreType.DMA,
        pltpu.SemaphoreType.DMA,
        pltpu.VMEM_SHARED((N_NODES, D_AGG), jnp.float32),
    ],
)(_spmm_body)


_BLK = 1000
_GRID = N_NODES // _BLK


def _dinv_block(p0, p1):
    deg = p0 + p1 + 1.0
    return jnp.where(deg > 0, lax.rsqrt(deg), 0.0)


def _prep_body(p0_ref, p1_ref, x_ref, xs_ref):
    dinv = _dinv_block(p0_ref[...], p1_ref[...])
    xs_ref[...] = x_ref[...] * dinv


def _layer1_body(z0_ref, z1_ref, xs_ref, p0_ref, p1_ref, w1_ref, b1_ref,
                 w2_ref, out_ref):
    dinv = _dinv_block(p0_ref[...], p1_ref[...])
    y1 = (z0_ref[...] + z1_ref[...] + xs_ref[...]) * dinv
    h1 = jnp.maximum(
        jnp.dot(y1, w1_ref[...], preferred_element_type=jnp.float32)
        + b1_ref[...],
        0.0,
    )
    out_ref[...] = (
        jnp.dot(h1, w2_ref[...], preferred_element_type=jnp.float32) * dinv
    )


def _layer2_body(z0_ref, z1_ref, xs_ref, p0_ref, p1_ref, b2_ref, wfc_ref,
                 bfc_ref, out_ref):
    dinv = _dinv_block(p0_ref[...], p1_ref[...])
    y2 = (z0_ref[...] + z1_ref[...] + xs_ref[...]) * dinv
    h2 = jnp.maximum(y2 + b2_ref[...], 0.0)
    out_ref[...] = (
        jnp.dot(h2, wfc_ref[...], preferred_element_type=jnp.float32)
        + bfc_ref[...]
    )


def _row_spec(width):
    return pl.BlockSpec((_BLK, width), lambda i: (i, 0))


def _full_spec(shape):
    return pl.BlockSpec(shape, lambda i: (0, 0))


_prep_call = pl.pallas_call(
    _prep_body,
    grid=(_GRID,),
    in_specs=[_row_spec(1), _row_spec(1), _row_spec(D_AGG)],
    out_specs=_row_spec(D_AGG),
    out_shape=jax.ShapeDtypeStruct((N_NODES, D_AGG), jnp.float32),
)

_layer1_call = pl.pallas_call(
    _layer1_body,
    grid=(_GRID,),
    in_specs=[
        _row_spec(D_AGG), _row_spec(D_AGG), _row_spec(D_AGG),
        _row_spec(1), _row_spec(1),
        _full_spec((128, 256)), _full_spec((1, 256)), _full_spec((256, 128)),
    ],
    out_specs=_row_spec(D_AGG),
    out_shape=jax.ShapeDtypeStruct((N_NODES, D_AGG), jnp.float32),
)

_layer2_call = pl.pallas_call(
    _layer2_body,
    grid=(_GRID,),
    in_specs=[
        _row_spec(D_AGG), _row_spec(D_AGG), _row_spec(D_AGG),
        _row_spec(1), _row_spec(1),
        _full_spec((1, 128)), _full_spec((128, 1)), _full_spec((1, 1)),
    ],
    out_specs=_row_spec(1),
    out_shape=jax.ShapeDtypeStruct((N_NODES, 1), jnp.float32),
)


def kernel(x, edge_index, edge_attr, W1, b1, W2, b2, Wfc, bfc):
    src = edge_index[0].reshape(NW, NCHUNK, CHUNK)
    dst = edge_index[1].reshape(NW, NCHUNK, CHUNK)
    ew = edge_attr.reshape(NW, NCHUNK, CHUNK)

    degp = _deg_call(dst, ew)
    p0 = degp[:N_NODES].reshape(N_NODES, 1)
    p1 = degp[N_NODES:].reshape(N_NODES, 1)

    xs1 = _prep_call(p0, p1, x)
    z = _spmm_call(src, dst, ew, xs1)
    xs2 = _layer1_call(
        z[:N_NODES], z[N_NODES:], xs1, p0, p1, W1, b1.reshape(1, -1), W2
    )
    z2 = _spmm_call(src, dst, ew, xs2)
    out = _layer2_call(
        z2[:N_NODES], z2[N_NODES:], xs2, p0, p1,
        b2.reshape(1, -1), Wfc, bfc.reshape(1, 1),
    )
    return out


# speedup vs baseline: 24.8120x; 1.0449x over previous
import functools

import jax
import jax.numpy as jnp
from jax import lax
from jax.experimental import pallas as pl
from jax.experimental.pallas import tpu as pltpu
from jax.experimental.pallas import tpu_sc as plsc

N_NODES = 10000
D_AGG = 128
E_TOTAL = 320000

NC, NS = 2, 16
NW = NC * NS
EPW = E_TOTAL // NW
CHUNK = 80
NCHUNK = EPW // CHUNK
ROWS_PT = N_NODES // NS

_MESH = plsc.VectorSubcoreMesh(core_axis_name="c", subcore_axis_name="s")


def _deg_body(dst_h, ew_h, out_h, dstv, ewv, zer, dsem, acc):
    c = lax.axis_index("c")
    s = lax.axis_index("s")
    wid = s * NC + c

    pltpu.sync_copy(dst_h.at[wid], dstv)
    pltpu.sync_copy(ew_h.at[wid], ewv)

    def zinit(i, carry):
        zer[pl.ds(pl.multiple_of(i * 16, 16), 16)] = jnp.zeros((16,), jnp.float32)
        return carry

    lax.fori_loop(0, 128, zinit, 0)

    @pl.when(s < 4)
    def _():
        pltpu.sync_copy(zer, acc.at[pl.ds(s * 2048, 2048)])

    @pl.when(s == 4)
    def _():
        pltpu.sync_copy(zer.at[pl.ds(0, 1808)], acc.at[pl.ds(8192, 1808)])

    plsc.subcore_barrier()

    def chunk_body(k, carry):
        pltpu.sync_copy(ewv.at[k], acc.at[dstv.at[k]], add=True)
        return carry

    lax.fori_loop(0, NCHUNK, chunk_body, 0)
    plsc.subcore_barrier()

    @pl.when(s < 4)
    def _():
        pltpu.sync_copy(acc.at[pl.ds(s * 2048, 2048)], zer)
        pltpu.sync_copy(zer, out_h.at[pl.ds(c * N_NODES + s * 2048, 2048)])

    @pl.when(s == 4)
    def _():
        pltpu.sync_copy(acc.at[pl.ds(8192, 1808)], zer.at[pl.ds(0, 1808)])
        pltpu.sync_copy(
            zer.at[pl.ds(0, 1808)], out_h.at[pl.ds(c * N_NODES + 8192, 1808)]
        )


_deg_call = functools.partial(
    pl.kernel,
    out_type=jax.ShapeDtypeStruct((2 * N_NODES,), jnp.float32),
    mesh=_MESH,
    scratch_types=[
        pltpu.VMEM((NCHUNK, CHUNK), jnp.int32),
        pltpu.VMEM((NCHUNK, CHUNK), jnp.float32),
        pltpu.VMEM((2048,), jnp.float32),
        pltpu.SemaphoreType.DMA,
        pltpu.VMEM_SHARED((N_NODES,), jnp.float32),
    ],
)(_deg_body)


def _spmm_body(src_h, dst_h, ew_h, xs_h, out_h, srcv, dstv, ewv,
               rows0, rows1, rows2, gsem0, gsem1, gsem2, ssem0, ssem1, ssem2,
               isem0, isem1, isem2, acc):
    c = lax.axis_index("c")
    s = lax.axis_index("s")
    wid = s * NC + c

    def zrow(e, carry):
        for j in range(8):
            rows0[e, pl.ds(j * 16, 16)] = jnp.zeros((16,), jnp.float32)
        return carry

    lax.fori_loop(0, 16, zrow, 0)
    base_r = s * 624
    nblk = 39 + jnp.where(s == 15, 1, 0)

    def zcopy(q, carry):
        pltpu.sync_copy(
            rows0.at[pl.ds(0, 16)], acc.at[pl.ds(base_r + q * 16, 16)]
        )
        return carry

    lax.fori_loop(0, nblk, zcopy, 0)
    plsc.subcore_barrier()

    def idx_load(k, j, sem):
        pltpu.async_copy(src_h.at[wid, k], srcv.at[j], sem)
        pltpu.async_copy(dst_h.at[wid, k], dstv.at[j], sem)
        pltpu.async_copy(ew_h.at[wid, k], ewv.at[j], sem)

    def idx_drain(j, sem):
        pltpu.make_async_copy(src_h.at[0, 0], srcv.at[j], sem).wait()
        pltpu.make_async_copy(dst_h.at[0, 0], dstv.at[j], sem).wait()
        pltpu.make_async_copy(ew_h.at[0, 0], ewv.at[j], sem).wait()

    def gather(j, rows, sem):
        return pltpu.async_copy(xs_h.at[srcv.at[j]], rows, sem)

    def scatter(j, rows, sem):
        return pltpu.async_copy(rows, acc.at[dstv.at[j]], sem, add=True)

    def drain(rows, sem):
        pltpu.make_async_copy(xs_h.at[pl.ds(0, CHUNK)], rows, sem).wait()

    def scale(rows, j):
        def group_body(g, inner):
            gbase = pl.multiple_of(g * 16, 16)
            wvec = ewv[j, pl.ds(gbase, 16)]
            for i in range(16):
                w = wvec[i]
                e = gbase + i
                for jj in range(8):
                    rows[e, pl.ds(jj * 16, 16)] = (
                        rows[e, pl.ds(jj * 16, 16)] * w
                    )
            return inner

        lax.fori_loop(0, CHUNK // 16, group_body, 0)

    idx_load(0, 0, isem0)
    idx_load(1, 1, isem1)
    idx_load(2, 2, isem2)

    def triad(t, carry):
        a = 3 * t
        idx_drain(0, isem0)
        g0 = gather(0, rows0, gsem0)
        idx_drain(1, isem1)
        g1 = gather(1, rows1, gsem1)
        idx_drain(2, isem2)
        g2 = gather(2, rows2, gsem2)
        g0.wait()
        scale(rows0, 0)
        s0 = scatter(0, rows0, ssem0)
        g1.wait()
        scale(rows1, 1)
        s1 = scatter(1, rows1, ssem1)
        g2.wait()
        scale(rows2, 2)
        s2 = scatter(2, rows2, ssem2)
        s0.wait()
        idx_load(a + 3, 0, isem0)
        s1.wait()

        @pl.when(a + 4 < NCHUNK)
        def _():
            idx_load(a + 4, 1, isem1)

        s2.wait()

        @pl.when(a + 5 < NCHUNK)
        def _():
            idx_load(a + 5, 2, isem2)

        return carry

    lax.fori_loop(0, NCHUNK // 3, triad, 0)
    idx_drain(0, isem0)
    g0 = gather(0, rows0, gsem0)
    idx_drain(1, isem1)
    g1 = gather(1, rows1, gsem1)
    g0.wait()
    scale(rows0, 0)
    s0 = scatter(0, rows0, ssem0)
    g1.wait()
    scale(rows1, 1)
    s1 = scatter(1, rows1, ssem1)
    s0.wait()
    s1.wait()
    plsc.subcore_barrier()

    out_base = c * N_NODES + s * 624
    pltpu.sync_copy(acc.at[pl.ds(s * 624, 624)], out_h.at[pl.ds(out_base, 624)])

    @pl.when(s == 15)
    def _():
        pltpu.sync_copy(
            acc.at[pl.ds(9984, 16)], out_h.at[pl.ds(c * N_NODES + 9984, 16)]
        )


_spmm_call = functools.partial(
    pl.kernel,
    out_type=jax.ShapeDtypeStruct((2 * N_NODES, D_AGG), jnp.float32),
    mesh=_MESH,
    scratch_types=[
        pltpu.VMEM((3, CHUNK), jnp.int32),
        pltpu.VMEM((3, CHUNK), jnp.int32),
        pltpu.VMEM((3, CHUNK), jnp.float32),
        pltpu.VMEM((CHUNK, D_AGG), jnp.float32),
        pltpu.VMEM((CHUNK, D_AGG), jnp.float32),
        pltpu.VMEM((CHUNK, D_AGG), jnp.float32),
        pltpu.SemaphoreType.DMA,
        pltpu.SemaphoreType.DMA,
        pltpu.SemaphoreType.DMA,
        pltpu.SemaphoreType.DMA,
        pltpu.SemaphoreType.DMA,
        pltpu.SemaphoreType.DMA,
        pltpu.SemaphoreType.DMA,
        pltpu.SemaphoreType.DMA,
        pltpu.SemaphoreType.DMA,
        pltpu.VMEM_SHARED((N_NODES, D_AGG), jnp.float32),
    ],
)(_spmm_body)


_BLK = 1000
_GRID = N_NODES // _BLK


def _dinv_block(p0, p1):
    deg = p0 + p1 + 1.0
    return jnp.where(deg > 0, lax.rsqrt(deg), 0.0)


def _prep_body(p0_ref, p1_ref, x_ref, xs_ref):
    dinv = _dinv_block(p0_ref[...], p1_ref[...])
    xs_ref[...] = x_ref[...] * dinv


def _layer1_body(z0_ref, z1_ref, xs_ref, p0_ref, p1_ref, w1_ref, b1_ref,
                 w2_ref, out_ref):
    dinv = _dinv_block(p0_ref[...], p1_ref[...])
    y1 = (z0_ref[...] + z1_ref[...] + xs_ref[...]) * dinv
    h1 = jnp.maximum(
        jnp.dot(y1, w1_ref[...], preferred_element_type=jnp.float32)
        + b1_ref[...],
        0.0,
    )
    out_ref[...] = (
        jnp.dot(h1, w2_ref[...], preferred_element_type=jnp.float32) * dinv
    )


def _layer2_body(z0_ref, z1_ref, xs_ref, p0_ref, p1_ref, b2_ref, wfc_ref,
                 bfc_ref, out_ref):
    dinv = _dinv_block(p0_ref[...], p1_ref[...])
    y2 = (z0_ref[...] + z1_ref[...] + xs_ref[...]) * dinv
    h2 = jnp.maximum(y2 + b2_ref[...], 0.0)
    out_ref[...] = (
        jnp.dot(h2, wfc_ref[...], preferred_element_type=jnp.float32)
        + bfc_ref[...]
    )


def _row_spec(width):
    return pl.BlockSpec((_BLK, width), lambda i: (i, 0))


def _full_spec(shape):
    return pl.BlockSpec(shape, lambda i: (0, 0))


_prep_call = pl.pallas_call(
    _prep_body,
    grid=(_GRID,),
    in_specs=[_row_spec(1), _row_spec(1), _row_spec(D_AGG)],
    out_specs=_row_spec(D_AGG),
    out_shape=jax.ShapeDtypeStruct((N_NODES, D_AGG), jnp.float32),
)

_layer1_call = pl.pallas_call(
    _layer1_body,
    grid=(_GRID,),
    in_specs=[
        _row_spec(D_AGG), _row_spec(D_AGG), _row_spec(D_AGG),
        _row_spec(1), _row_spec(1),
        _full_spec((128, 256)), _full_spec((1, 256)), _full_spec((256, 128)),
    ],
    out_specs=_row_spec(D_AGG),
    out_shape=jax.ShapeDtypeStruct((N_NODES, D_AGG), jnp.float32),
)

_layer2_call = pl.pallas_call(
    _layer2_body,
    grid=(_GRID,),
    in_specs=[
        _row_spec(D_AGG), _row_spec(D_AGG), _row_spec(D_AGG),
        _row_spec(1), _row_spec(1),
        _full_spec((1, 128)), _full_spec((128, 1)), _full_spec((1, 1)),
    ],
    out_specs=_row_spec(1),
    out_shape=jax.ShapeDtypeStruct((N_NODES, 1), jnp.float32),
)


def kernel(x, edge_index, edge_attr, W1, b1, W2, b2, Wfc, bfc):
    src = edge_index[0].reshape(NW, NCHUNK, CHUNK)
    dst = edge_index[1].reshape(NW, NCHUNK, CHUNK)
    ew = edge_attr.reshape(NW, NCHUNK, CHUNK)

    degp = _deg_call(dst, ew)
    p0 = degp[:N_NODES].reshape(N_NODES, 1)
    p1 = degp[N_NODES:].reshape(N_NODES, 1)

    xs1 = _prep_call(p0, p1, x)
    z = _spmm_call(src, dst, ew, xs1)
    xs2 = _layer1_call(
        z[:N_NODES], z[N_NODES:], xs1, p0, p1, W1, b1.reshape(1, -1), W2
    )
    z2 = _spmm_call(src, dst, ew, xs2)
    out = _layer2_call(
        z2[:N_NODES], z2[N_NODES:], xs2, p0, p1,
        b2.reshape(1, -1), Wfc, bfc.reshape(1, 1),
    )
    return out
